# baseline (speedup 1.0000x reference)
import jax
import jax.numpy as jnp
from jax import lax
from jax.experimental import pallas as pl
from jax.experimental.pallas import tpu as pltpu
from jax.experimental.pallas import tpu_sc as plsc

N = 10000
D = 128
E = 320000
NC, NS = 2, 16
NW = NC * NS
NP = 10240
RPT = NP // NS
CHUNK = 128
IB = 16
TB = 160
EP = TB * IB * CHUNK
NBT = TB // NW
B0 = 9
DUMP = NP - 1

_mesh = plsc.VectorSubcoreMesh(core_axis_name="c", subcore_axis_name="s")


def _sc_aggregate(x_pad, src, dst, z128):
    def body(x_hbm, src_hbm, dst_hbm, z128_hbm, agg_hbm,
             idxs_v, idxd_v, rows_v, sem, acc_sh):
        c = lax.axis_index("c")
        s = lax.axis_index("s")
        row0 = s * RPT
        pltpu.sync_copy(z128_hbm.at[pl.ds(row0, RPT)],
                        acc_sh.at[pl.ds(row0, RPT)])
        plsc.subcore_barrier()

        nblk = jnp.where(c == 0, B0, 2 * NBT - B0)
        base = jnp.where(c == 0, s * B0, NS * B0 + s * (2 * NBT - B0))

        def block(b, carry):
            pltpu.sync_copy(src_hbm.at[base + b], idxs_v)
            pltpu.sync_copy(dst_hbm.at[base + b], idxd_v)
            for j in range(IB):
                pltpu.async_copy(x_hbm.at[idxs_v.at[j]], rows_v, sem).wait()
                pltpu.sync_copy(rows_v, acc_sh.at[idxd_v.at[j]], add=True)
            return carry

        lax.fori_loop(0, nblk, block, 0)
        plsc.subcore_barrier()
        pltpu.sync_copy(acc_sh.at[pl.ds(row0, RPT)],
                        agg_hbm.at[c, pl.ds(row0, RPT)])

    return pl.kernel(
        body,
        out_type=[jax.ShapeDtypeStruct((NC, NP, D), jnp.float32)],
        mesh=_mesh,
        scratch_types=[
            pltpu.VMEM((IB, CHUNK), jnp.int32),
            pltpu.VMEM((IB, CHUNK), jnp.int32),
            pltpu.VMEM((CHUNK, D), jnp.float32),
            pltpu.SemaphoreType.DMA,
            pltpu.VMEM_SHARED((NP, D), jnp.float32),
        ],
    )(x_pad, src, dst, z128)[0]


def _sc_aggregate_deg(x_pad, src, dst, z128, ones):
    def body(x_hbm, src_hbm, dst_hbm, z128_hbm, ones_hbm, out_hbm,
             idxs_v, idxd_v, rows_v, ones_v, sem, acc_sh):
        c = lax.axis_index("c")
        s = lax.axis_index("s")
        row0 = s * RPT
        pltpu.sync_copy(z128_hbm.at[pl.ds(row0, RPT)],
                        acc_sh.at[pl.ds(row0, RPT)])
        pltpu.sync_copy(ones_hbm, ones_v)
        plsc.subcore_barrier()

        base = s * (TB // NS)
        nblk_g = jnp.where(c == 0, TB // NS, 0)
        nblk_d = jnp.where(c == 0, 0, TB // NS)

        def gblock(b, carry):
            pltpu.sync_copy(src_hbm.at[base + b], idxs_v)
            pltpu.sync_copy(dst_hbm.at[base + b], idxd_v)
            for j in range(IB):
                pltpu.async_copy(x_hbm.at[idxs_v.at[j]], rows_v, sem).wait()
                pltpu.sync_copy(rows_v, acc_sh.at[idxd_v.at[j]], add=True)
            return carry

        def dblock(b, carry):
            pltpu.sync_copy(dst_hbm.at[base + b], idxd_v)
            for j in range(IB):
                pltpu.sync_copy(ones_v, acc_sh.at[idxd_v.at[j]], add=True)
            return carry

        lax.fori_loop(0, nblk_g, gblock, 0)
        lax.fori_loop(0, nblk_d, dblock, 0)
        plsc.subcore_barrier()
        pltpu.sync_copy(acc_sh.at[pl.ds(row0, RPT)],
                        out_hbm.at[c, pl.ds(row0, RPT)])

    return pl.kernel(
        body,
        out_type=[jax.ShapeDtypeStruct((NC, NP, D), jnp.float32)],
        mesh=_mesh,
        scratch_types=[
            pltpu.VMEM((IB, CHUNK), jnp.int32),
            pltpu.VMEM((IB, CHUNK), jnp.int32),
            pltpu.VMEM((CHUNK, D), jnp.float32),
            pltpu.VMEM((CHUNK, D), jnp.float32),
            pltpu.SemaphoreType.DMA,
            pltpu.VMEM_SHARED((NP, D), jnp.float32),
        ],
    )(x_pad, src, dst, z128, ones)[0]


def _make_tc_combine(relu, agg_first_only):
    def body(ap_ref, dp_ref, x_ref, wl_ref, bl_ref, wr_ref, o_ref):
        agg = ap_ref[0] if agg_first_only else ap_ref[0] + ap_ref[1]
        deg = jnp.maximum(dp_ref[1, :, 0], 1.0)
        mean = agg / deg[:, None]
        out = (jnp.dot(mean, wl_ref[...], preferred_element_type=jnp.float32)
               + bl_ref[...]
               + jnp.dot(x_ref[...], wr_ref[...],
                         preferred_element_type=jnp.float32))
        if relu:
            out = jnp.maximum(out, 0.0)
        o_ref[...] = out

    BN = 1024
    return pl.pallas_call(
        body,
        grid=(NP // BN,),
        in_specs=[
            pl.BlockSpec((NC, BN, D), lambda i: (0, i, 0)),
            pl.BlockSpec((NC, BN, D), lambda i: (0, i, 0)),
            pl.BlockSpec((BN, D), lambda i: (i, 0)),
            pl.BlockSpec((D, D), lambda i: (0, 0)),
            pl.BlockSpec((1, D), lambda i: (0, 0)),
            pl.BlockSpec((D, D), lambda i: (0, 0)),
        ],
        out_specs=pl.BlockSpec((BN, D), lambda i: (i, 0)),
        out_shape=jax.ShapeDtypeStruct((NP, D), jnp.float32),
    )


_tc_relu = _make_tc_combine(True, True)
_tc_plain = _make_tc_combine(False, False)


def kernel(x, edge_index, W_l1, b_l1, W_r1, W_l2, b_l2, W_r2):
    src = jnp.concatenate(
        [edge_index[0].astype(jnp.int32),
         jnp.zeros((EP - E,), jnp.int32)]).reshape(TB, IB, CHUNK)
    dst = jnp.concatenate(
        [edge_index[1].astype(jnp.int32),
         jnp.full((EP - E,), DUMP, jnp.int32)]).reshape(TB, IB, CHUNK)
    x_pad = jnp.pad(x, ((0, NP - N), (0, 0)))
    z128 = jnp.zeros((NP, D), jnp.float32)
    ones = jnp.ones((CHUNK, D), jnp.float32)

    agg1d = _sc_aggregate_deg(x_pad, src, dst, z128, ones)
    h1 = _tc_relu(agg1d, agg1d, x_pad, W_l1, b_l1.reshape(1, D), W_r1)
    agg2 = _sc_aggregate(h1, src, dst, z128)
    out = _tc_plain(agg2, agg1d, h1, W_l2, b_l2.reshape(1, D), W_r2)
    return out[:N]


# speedup vs baseline: 1.2011x; 1.2011x over previous
import jax
import jax.numpy as jnp
from jax import lax
from jax.experimental import pallas as pl
from jax.experimental.pallas import tpu as pltpu
from jax.experimental.pallas import tpu_sc as plsc

N = 10000
D = 128
E = 320000
NC, NS = 2, 16
NW = NC * NS
NP = 10240
RPT = NP // NS
CHUNK = 128
IB = 16
TB = 160
EP = TB * IB * CHUNK
NBT = TB // NW
B0 = 9
DUMP = NP - 1

_mesh = plsc.VectorSubcoreMesh(core_axis_name="c", subcore_axis_name="s")


def _sc_aggregate(x_pad, src, dst, z128):
    def body(x_hbm, src_hbm, dst_hbm, z128_hbm, agg_hbm,
             idxs_v, idxd_v, rows_v, sem, acc_sh):
        c = lax.axis_index("c")
        s = lax.axis_index("s")
        row0 = s * RPT
        pltpu.sync_copy(z128_hbm.at[pl.ds(row0, RPT)],
                        acc_sh.at[pl.ds(row0, RPT)])
        plsc.subcore_barrier()

        nblk = jnp.where(c == 0, B0, 2 * NBT - B0)
        base = jnp.where(c == 0, s * B0, NS * B0 + s * (2 * NBT - B0))

        def block(b, carry):
            pltpu.sync_copy(src_hbm.at[base + b], idxs_v)
            pltpu.sync_copy(dst_hbm.at[base + b], idxd_v)
            for j in range(IB):
                pltpu.async_copy(x_hbm.at[idxs_v.at[j]], rows_v, sem).wait()
                pltpu.sync_copy(rows_v, acc_sh.at[idxd_v.at[j]], add=True)
            return carry

        lax.fori_loop(0, nblk, block, 0)
        plsc.subcore_barrier()
        pltpu.sync_copy(acc_sh.at[pl.ds(row0, RPT)],
                        agg_hbm.at[c, pl.ds(row0, RPT)])

    return pl.kernel(
        body,
        out_type=[jax.ShapeDtypeStruct((NC, NP, D), jnp.float32)],
        mesh=_mesh,
        scratch_types=[
            pltpu.VMEM((IB, CHUNK), jnp.int32),
            pltpu.VMEM((IB, CHUNK), jnp.int32),
            pltpu.VMEM((CHUNK, D), jnp.float32),
            pltpu.SemaphoreType.DMA,
            pltpu.VMEM_SHARED((NP, D), jnp.float32),
        ],
    )(x_pad, src, dst, z128)[0]


def _sc_degree(dst, z128, ones):
    def body(dst_hbm, z128_hbm, ones_hbm, deg_hbm,
             idxd_v, ones_v, deg_sh):
        c = lax.axis_index("c")
        s = lax.axis_index("s")
        wid = s * NC + c
        row0 = s * RPT
        pltpu.sync_copy(z128_hbm.at[pl.ds(row0, RPT)],
                        deg_sh.at[pl.ds(row0, RPT)])
        pltpu.sync_copy(ones_hbm, ones_v)
        plsc.subcore_barrier()

        def block(b, carry):
            pltpu.sync_copy(dst_hbm.at[wid * NBT + b], idxd_v)
            for j in range(IB):
                pltpu.sync_copy(ones_v, deg_sh.at[idxd_v.at[j]], add=True)
            return carry

        lax.fori_loop(0, NBT, block, 0)
        plsc.subcore_barrier()
        pltpu.sync_copy(deg_sh.at[pl.ds(row0, RPT)],
                        deg_hbm.at[c, pl.ds(row0, RPT)])

    return pl.kernel(
        body,
        out_type=[jax.ShapeDtypeStruct((NC, NP, D), jnp.float32)],
        mesh=_mesh,
        scratch_types=[
            pltpu.VMEM((IB, CHUNK), jnp.int32),
            pltpu.VMEM((CHUNK, D), jnp.float32),
            pltpu.VMEM_SHARED((NP, D), jnp.float32),
        ],
    )(dst, z128, ones)[0]


def _make_tc_combine(relu):
    def body(ap_ref, dp_ref, x_ref, wl_ref, bl_ref, wr_ref, o_ref):
        agg = ap_ref[0] + ap_ref[1]
        deg = jnp.maximum(dp_ref[0, :, 0] + dp_ref[1, :, 0], 1.0)
        mean = agg / deg[:, None]
        out = (jnp.dot(mean, wl_ref[...], preferred_element_type=jnp.float32)
               + bl_ref[...]
               + jnp.dot(x_ref[...], wr_ref[...],
                         preferred_element_type=jnp.float32))
        if relu:
            out = jnp.maximum(out, 0.0)
        o_ref[...] = out

    BN = 1024
    return pl.pallas_call(
        body,
        grid=(NP // BN,),
        in_specs=[
            pl.BlockSpec((NC, BN, D), lambda i: (0, i, 0)),
            pl.BlockSpec((NC, BN, D), lambda i: (0, i, 0)),
            pl.BlockSpec((BN, D), lambda i: (i, 0)),
            pl.BlockSpec((D, D), lambda i: (0, 0)),
            pl.BlockSpec((1, D), lambda i: (0, 0)),
            pl.BlockSpec((D, D), lambda i: (0, 0)),
        ],
        out_specs=pl.BlockSpec((BN, D), lambda i: (i, 0)),
        out_shape=jax.ShapeDtypeStruct((NP, D), jnp.float32),
    )


_tc_relu = _make_tc_combine(True)
_tc_plain = _make_tc_combine(False)


def kernel(x, edge_index, W_l1, b_l1, W_r1, W_l2, b_l2, W_r2):
    src = jnp.concatenate(
        [edge_index[0].astype(jnp.int32),
         jnp.zeros((EP - E,), jnp.int32)]).reshape(TB, IB, CHUNK)
    dst = jnp.concatenate(
        [edge_index[1].astype(jnp.int32),
         jnp.full((EP - E,), DUMP, jnp.int32)]).reshape(TB, IB, CHUNK)
    x_pad = jnp.pad(x, ((0, NP - N), (0, 0)))
    z128 = jnp.zeros((NP, D), jnp.float32)
    ones = jnp.ones((CHUNK, D), jnp.float32)

    degp = _sc_degree(dst, z128, ones)
    agg1 = _sc_aggregate(x_pad, src, dst, z128)
    h1 = _tc_relu(agg1, degp, x_pad, W_l1, b_l1.reshape(1, D), W_r1)
    agg2 = _sc_aggregate(h1, src, dst, z128)
    out = _tc_plain(agg2, degp, h1, W_l2, b_l2.reshape(1, D), W_r2)
    return out[:N]
